# baseline (speedup 1.0000x reference)
import jax
import jax.numpy as jnp
from jax import lax
from jax.experimental import pallas as pl
from jax.experimental.pallas import tpu as pltpu
from jax.experimental.pallas import tpu_sc as plsc

N = 10000
E = 320000

NC = 2
NS = 16
CH1 = 64
CH2 = 128
NPAD = 10240
EPT = NPAD
NCH1 = EPT // CH1
NCH2 = EPT // CH2
EP2 = NC * NS * EPT
RPT = NPAD // NS
DUMMY = NPAD - 8

_SC_PARAMS = pltpu.CompilerParams(
    use_tc_tiling_on_sc=False, needs_layout_passes=False)


def _lrelu_exp(s):
    return jnp.exp(jnp.where(s >= 0, s, 0.2 * s))


def _mm1_body(x_ref, w_ref, o_ref):
    o_ref[...] = jnp.dot(x_ref[...], w_ref[...],
                         preferred_element_type=jnp.float32)


def _tc_mm1(x2, w1e):
    return pl.pallas_call(
        _mm1_body,
        grid=(40,),
        in_specs=[
            pl.BlockSpec((512, 128), lambda i: (i, 0)),
            pl.BlockSpec((128, 96), lambda i: (0, 0)),
        ],
        out_specs=pl.BlockSpec((512, 96), lambda i: (i, 0)),
        out_shape=jax.ShapeDtypeStruct((2 * NPAD, 96), jnp.float32),
    )(x2, w1e)


def _sc1_body(h0, a0, s0, d0, h1, a1, s1, d1, zr,
              out, acc0, acc1,
              sb0, sb1, sb2, db0, db1, db2,
              hg0, hg1, hg2, ag0, ag1, ag2, msg,
              isem0, isem1, isem2, gsem0, gsem1, gsem2):
    c = lax.axis_index("c")
    s = lax.axis_index("s")
    tid = c * NS + s

    sb = (sb0, sb1, sb2)
    db = (db0, db1, db2)
    hg = (hg0, hg1, hg2)
    ag = (ag0, ag1, ag2)
    isem = (isem0, isem1, isem2)
    gsem = (gsem0, gsem1, gsem2)

    pltpu.sync_copy(zr, acc0.at[pl.ds(s * RPT, RPT)])
    pltpu.sync_copy(zr, acc1.at[pl.ds(s * RPT, RPT)])
    plsc.subcore_barrier()

    iota = lax.iota(jnp.int32, 16)
    eoff2 = lax.shift_right_logical(iota, 3)
    pat0 = [2 * part + eoff2 for part in range(4)]
    pat1 = [p + 8 for p in pat0]
    swap8 = jnp.bitwise_xor(iota, 8)
    maskf = jnp.where(iota < 8, 1.0, 0.0).astype(jnp.float32)

    def perm(v, idx):
        return v.at[idx].get(mode="promise_in_bounds")

    for hT, aT, sT, dT, acc in ((h0, a0, s0, d0, acc0),
                                (h1, a1, s1, d1, acc1)):
        def issue_idx(x, q, sT=sT, dT=dT):
            base = pl.multiple_of(tid * EPT + x * CH1, 8)
            pltpu.async_copy(sT.at[pl.ds(base, CH1)], sb[q], isem[q])
            pltpu.async_copy(dT.at[pl.ds(base, CH1)], db[q], isem[q])

        def launch_gather(q, sT=sT, dT=dT, hT=hT, aT=aT):
            pltpu.make_async_copy(sT.at[pl.ds(0, CH1)], sb[q], isem[q]).wait()
            pltpu.make_async_copy(dT.at[pl.ds(0, CH1)], db[q], isem[q]).wait()
            pltpu.async_copy(hT.at[sb[q]], hg[q], gsem[q])
            pltpu.async_copy(aT.at[db[q]], ag[q], gsem[q])

        def compute(q, hT=hT, aT=aT, acc=acc):
            pltpu.make_async_copy(hT.at[sb[q]], hg[q], gsem[q]).wait()
            pltpu.make_async_copy(aT.at[db[q]], ag[q], gsem[q]).wait()
            hgp, agp = hg[q], ag[q]

            def mr(j4, _):
                for u in range(4):
                    j = j4 * 4 + u
                    r0 = 2 * j
                    r1 = 2 * j + 1
                    v0 = hgp[r0, pl.ds(32, 16)]
                    v1 = hgp[r1, pl.ds(32, 16)]
                    a0v = agp[r0, pl.ds(0, 16)]
                    a1v = agp[r1, pl.ds(0, 16)]
                    asel = jnp.where(iota < 8, v0, v1)
                    adsel = jnp.where(iota < 8, a0v, a1v)
                    w2 = _lrelu_exp(asel + adsel)
                    for r, pats in ((r0, pat0), (r1, pat1)):
                        for half in range(2):
                            pw = hgp[r, pl.ds(half * 16, 16)]
                            ha, hb = plsc.unpack(
                                plsc.bitcast(pw, jnp.bfloat16),
                                format=plsc.PackFormat.INTERLEAVED)
                            msg[r, pl.ds(half * 32, 16)] = (
                                ha * perm(w2, pats[2 * half]))
                            msg[r, pl.ds(half * 32 + 16, 16)] = (
                                hb * perm(w2, pats[2 * half + 1]))
                    msg[r0, pl.ds(64, 16)] = w2 * maskf
                    msg[r1, pl.ds(64, 16)] = perm(w2, swap8) * maskf
                return ()

            lax.fori_loop(0, CH1 // 8, mr, ())
            pltpu.sync_copy(msg, acc.at[db[q]], add=True)

        issue_idx(0, 0)
        issue_idx(1, 1)
        launch_gather(0)
        launch_gather(1)
        issue_idx(2, 2)

        def step(k, _):
            for q in range(3):
                x = 3 * k + q

                @pl.when(x + 2 < NCH1)
                def _(q=q):
                    launch_gather((q + 2) % 3)

                @pl.when(x < NCH1)
                def _(q=q):
                    compute(q)

                @pl.when(x + 3 < NCH1)
                def _(x=x, q=q):
                    issue_idx(x + 3, q)

            return ()

        lax.fori_loop(0, (NCH1 + 2) // 3, step, ())

    plsc.subcore_barrier()
    pltpu.sync_copy(acc0.at[pl.ds(s * RPT, RPT)],
                    out.at[c, 0, pl.ds(s * RPT, RPT)])
    pltpu.sync_copy(acc1.at[pl.ds(s * RPT, RPT)],
                    out.at[c, 1, pl.ds(s * RPT, RPT)])


def _sc_layer1(h0, a0, s0, d0, h1, a1, s1, d1, zr):
    mesh = plsc.VectorSubcoreMesh(core_axis_name="c", subcore_axis_name="s",
                                  num_cores=NC, num_subcores=NS)
    f = pl.kernel(
        _sc1_body,
        out_type=jax.ShapeDtypeStruct((NC, 2, NPAD, 80), jnp.float32),
        mesh=mesh,
        compiler_params=_SC_PARAMS,
        scratch_types=(
            [pltpu.VMEM_SHARED((NPAD, 80), jnp.float32)] * 2
            + [pltpu.VMEM((CH1,), jnp.int32)] * 6
            + [pltpu.VMEM((CH1, 48), jnp.float32)] * 3
            + [pltpu.VMEM((CH1, 16), jnp.float32)] * 3
            + [pltpu.VMEM((CH1, 80), jnp.float32)]
            + [pltpu.SemaphoreType.DMA] * 6
        ),
    )
    return f(h0, a0, s0, d0, h1, a1, s1, d1, zr)


def _tc2_body(p0, p1, he, ad, b1r, rr, w2e, o_ref):
    ssum = p0[...] + p1[...]
    msg = ssum[:, 0:64]
    den = ssum[:, 64:72]
    asrc = he[:, 64:72]
    adst = ad[:, 0:8]
    wself = _lrelu_exp(asrc + adst)
    h1 = he[:, 0:64]
    wexp = jnp.dot(wself, rr[...], preferred_element_type=jnp.float32)
    dexp = jnp.dot(den + wself, rr[...], preferred_element_type=jnp.float32)
    out1 = (msg + h1 * wexp) / dexp + b1r[0:1, :]
    act = jnp.where(out1 > 0, out1, jnp.exp(out1) - 1.0)
    o_ref[...] = jnp.dot(act, w2e[...], preferred_element_type=jnp.float32)


def _tc_combine2(p0, p1, he, ad, b1r, rr, w2e):
    return pl.pallas_call(
        _tc2_body,
        grid=(40,),
        in_specs=[
            pl.BlockSpec((512, 80), lambda i: (i, 0)),
            pl.BlockSpec((512, 80), lambda i: (i, 0)),
            pl.BlockSpec((512, 80), lambda i: (i, 0)),
            pl.BlockSpec((512, 16), lambda i: (i, 0)),
            pl.BlockSpec((8, 64), lambda i: (0, 0)),
            pl.BlockSpec((8, 64), lambda i: (0, 0)),
            pl.BlockSpec((64, 32), lambda i: (0, 0)),
        ],
        out_specs=pl.BlockSpec((512, 32), lambda i: (i, 0)),
        out_shape=jax.ShapeDtypeStruct((2 * NPAD, 32), jnp.float32),
    )(p0, p1, he, ad, b1r, rr, w2e)


def _sc2_body(h0, a0, s0, d0, h1, a1, s1, d1, zr,
              out, acc0, acc1, av0, av1,
              sb0, sb1, sb2, sb3, db0, db1, db2, db3,
              hg0, hg1, hg2, hg3, msg, wbuf,
              isem0, isem1, isem2, isem3, gsem0, gsem1, gsem2, gsem3):
    c = lax.axis_index("c")
    s = lax.axis_index("s")
    tid = c * NS + s

    sb = (sb0, sb1, sb2, sb3)
    db = (db0, db1, db2, db3)
    hg = (hg0, hg1, hg2, hg3)
    isem = (isem0, isem1, isem2, isem3)
    gsem = (gsem0, gsem1, gsem2, gsem3)

    pltpu.sync_copy(zr, acc0.at[pl.ds(s * RPT, RPT)])
    pltpu.sync_copy(zr, acc1.at[pl.ds(s * RPT, RPT)])
    pltpu.sync_copy(a0, av0)
    pltpu.sync_copy(a1, av1)
    plsc.subcore_barrier()

    iota = lax.iota(jnp.int32, 16)
    c16 = jnp.full((16,), 16, jnp.int32)
    mask0 = jnp.where(iota < 1, 1.0, 0.0).astype(jnp.float32)

    for hT, sT, dT, av, acc in ((h0, s0, d0, av0, acc0),
                                (h1, s1, d1, av1, acc1)):
        def issue_idx(x, q, sT=sT, dT=dT):
            base = pl.multiple_of(tid * EPT + x * CH2, 8)
            pltpu.async_copy(sT.at[pl.ds(base, CH2)], sb[q], isem[q])
            pltpu.async_copy(dT.at[pl.ds(base, CH2)], db[q], isem[q])

        def launch_gather(q, sT=sT, dT=dT, hT=hT):
            pltpu.make_async_copy(sT.at[pl.ds(0, CH2)], sb[q], isem[q]).wait()
            pltpu.make_async_copy(dT.at[pl.ds(0, CH2)], db[q], isem[q]).wait()
            pltpu.async_copy(hT.at[sb[q]], hg[q], gsem[q])

        def compute(q, hT=hT, av=av, acc=acc):
            pltpu.make_async_copy(hT.at[sb[q]], hg[q], gsem[q]).wait()
            hgp, dbp = hg[q], db[q]

            def wj(j, _):
                erow = iota + 16 * j
                asrc = plsc.load_gather(hgp, [erow, c16])
                didx = dbp[pl.ds(j * 16, 16)]
                adst = plsc.load_gather(av, [didx])
                wbuf[pl.ds(j * 16, 16)] = _lrelu_exp(asrc + adst)
                return ()

            lax.fori_loop(0, CH2 // 16, wj, ())

            def mr(r8, _):
                for u in range(8):
                    r = r8 * 8 + u
                    rv = jnp.full((16,), 0, jnp.int32) + r
                    wv = plsc.load_gather(wbuf, [rv])
                    msg[r, pl.ds(0, 16)] = hgp[r, pl.ds(0, 16)] * wv
                    msg[r, pl.ds(16, 16)] = wv * mask0
                return ()

            lax.fori_loop(0, CH2 // 8, mr, ())
            pltpu.sync_copy(msg, acc.at[dbp], add=True)

        issue_idx(0, 0)
        issue_idx(1, 1)
        launch_gather(0)
        launch_gather(1)
        issue_idx(2, 2)
        issue_idx(3, 3)

        def step(k, _):
            for q in range(4):
                x = 4 * k + q

                @pl.when(x + 2 < NCH2)
                def _(q=q):
                    launch_gather((q + 2) % 4)

                compute(q)

                @pl.when(x + 4 < NCH2)
                def _(x=x, q=q):
                    issue_idx(x + 4, q)

            return ()

        lax.fori_loop(0, NCH2 // 4, step, ())

    plsc.subcore_barrier()
    pltpu.sync_copy(acc0.at[pl.ds(s * RPT, RPT)],
                    out.at[c, 0, pl.ds(s * RPT, RPT)])
    pltpu.sync_copy(acc1.at[pl.ds(s * RPT, RPT)],
                    out.at[c, 1, pl.ds(s * RPT, RPT)])


def _sc_layer2(h0, a0, s0, d0, h1, a1, s1, d1, zr):
    mesh = plsc.VectorSubcoreMesh(core_axis_name="c", subcore_axis_name="s",
                                  num_cores=NC, num_subcores=NS)
    f = pl.kernel(
        _sc2_body,
        out_type=jax.ShapeDtypeStruct((NC, 2, NPAD, 32), jnp.float32),
        mesh=mesh,
        compiler_params=_SC_PARAMS,
        scratch_types=(
            [pltpu.VMEM_SHARED((NPAD, 32), jnp.float32)] * 2
            + [pltpu.VMEM((NPAD,), jnp.float32)] * 2
            + [pltpu.VMEM((CH2,), jnp.int32)] * 8
            + [pltpu.VMEM((CH2, 32), jnp.float32)] * 4
            + [pltpu.VMEM((CH2, 32), jnp.float32)]
            + [pltpu.VMEM((CH2,), jnp.float32)]
            + [pltpu.SemaphoreType.DMA] * 8
        ),
    )
    return f(h0, a0, s0, d0, h1, a1, s1, d1, zr)


def _tc3_body(qy0, qy1, qz0, qz1, hy, hz, b2r, ly_ref, lz_ref, omc_ref):
    def node_out(q0, q1, he):
        ssum = q0[...] + q1[...]
        msg = ssum[:, 0:16]
        den = ssum[:, 16:17]
        asrc = he[:, 16:17]
        adst = he[:, 17:18]
        wself = _lrelu_exp(asrc + adst)
        h2 = he[:, 0:16]
        return (msg + h2 * wself) / (den + wself) + b2r[0:1, :]

    y = node_out(qy0, qy1, hy)
    z = node_out(qz0, qz1, hz)

    def logsm(v):
        m = jnp.max(v, axis=1, keepdims=True)
        return v - m - jnp.log(jnp.sum(jnp.exp(v - m), axis=1, keepdims=True))

    ly_ref[...] = logsm(y)
    lz_ref[...] = logsm(z)
    dot = jnp.sum(y * z, axis=1, keepdims=True)
    ny = jnp.maximum(jnp.sqrt(jnp.sum(y * y, axis=1, keepdims=True)), 1e-8)
    nz = jnp.maximum(jnp.sqrt(jnp.sum(z * z, axis=1, keepdims=True)), 1e-8)
    omc_ref[...] = 1.0 - dot / (ny * nz)


def _tc_final(qy0, qy1, qz0, qz1, hy, hz, b2r):
    return pl.pallas_call(
        _tc3_body,
        grid=(25,),
        in_specs=[
            pl.BlockSpec((400, 32), lambda i: (i, 0)),
            pl.BlockSpec((400, 32), lambda i: (i, 0)),
            pl.BlockSpec((400, 32), lambda i: (i, 0)),
            pl.BlockSpec((400, 32), lambda i: (i, 0)),
            pl.BlockSpec((400, 32), lambda i: (i, 0)),
            pl.BlockSpec((400, 32), lambda i: (i, 0)),
            pl.BlockSpec((8, 16), lambda i: (0, 0)),
        ],
        out_specs=[
            pl.BlockSpec((400, 16), lambda i: (i, 0)),
            pl.BlockSpec((400, 16), lambda i: (i, 0)),
            pl.BlockSpec((400, 1), lambda i: (i, 0)),
        ],
        out_shape=[
            jax.ShapeDtypeStruct((N, 16), jnp.float32),
            jax.ShapeDtypeStruct((N, 16), jnp.float32),
            jax.ShapeDtypeStruct((N, 1), jnp.float32),
        ],
    )(qy0, qy1, qz0, qz1, hy, hz, b2r)


@jax.jit
def kernel(x, edge_index, trans_x, trans_edge_index,
           W1, a1_src, a1_dst, b1, W2, a2_src, a2_dst, b2):
    f32 = jnp.float32
    ar64 = jnp.arange(64)
    A1s = jnp.zeros((64, 8), f32).at[ar64, ar64 // 8].set(a1_src.reshape(64))
    A1d = jnp.zeros((64, 8), f32).at[ar64, ar64 // 8].set(a1_dst.reshape(64))
    w1s = W1 @ A1s
    w1d = W1 @ A1d
    w1e = jnp.concatenate([W1, w1s, w1s, w1d, w1d], axis=1)

    zrow = jnp.zeros((NPAD - N, 128), f32)
    x2 = jnp.concatenate([x, zrow, trans_x, zrow], axis=0)
    y1 = _tc_mm1(x2, w1e)
    hext = y1[:, 0:80]
    adstt = y1[:, 80:96]

    epad = jnp.full((2, EP2 - E), DUMMY, jnp.int32)
    ei0 = jnp.concatenate([edge_index, epad], axis=1)
    ei1 = jnp.concatenate([trans_edge_index, epad], axis=1)
    s0, d0 = ei0[0], ei0[1]
    s1, d1 = ei1[0], ei1[1]

    hp = hext[:, 0:64].reshape(-1, 2, 2, 16).transpose(0, 1, 3, 2)
    packed = jax.lax.bitcast_convert_type(
        hp.astype(jnp.bfloat16), f32).reshape(-1, 32)
    tbl1 = jnp.concatenate([packed, hext[:, 64:80]], axis=1)
    zr1 = jnp.zeros((RPT, 80), f32)
    p = _sc_layer1(tbl1[:NPAD], adstt[:NPAD], s0, d0,
                   tbl1[NPAD:], adstt[NPAD:], s1, d1, zr1)

    w2e = jnp.concatenate(
        [W2, W2 @ a2_src.T, W2 @ a2_dst.T, jnp.zeros((64, 14), f32)], axis=1)
    rr = jnp.zeros((8, 64), f32).at[ar64 // 8, ar64].set(1.0)
    b1r = jnp.broadcast_to(b1, (8, 64))
    h2e = _tc_combine2(p[0].reshape(2 * NPAD, 80), p[1].reshape(2 * NPAD, 80),
                       hext, adstt, b1r, rr, w2e)

    av = h2e[:, 17]
    zr2 = jnp.zeros((RPT, 32), f32)
    q = _sc_layer2(h2e[:NPAD], av[:NPAD], s0, d0,
                   h2e[NPAD:], av[NPAD:], s1, d1, zr2)

    b2r = jnp.broadcast_to(b2, (8, 16))
    ly, lz, omc = _tc_final(q[0, 0], q[1, 0], q[0, 1], q[1, 1],
                            h2e[:NPAD], h2e[NPAD:], b2r)
    return (ly, omc.reshape(N), lz, ly, ly)


# speedup vs baseline: 1.0579x; 1.0579x over previous
import jax
import jax.numpy as jnp
from jax import lax
from jax.experimental import pallas as pl
from jax.experimental.pallas import tpu as pltpu
from jax.experimental.pallas import tpu_sc as plsc

N = 10000
E = 320000

NC = 2
NS = 16
CH1 = 64
CH2 = 128
NPAD = 10240
EPT = NPAD
NCH1 = EPT // CH1
NCH2 = EPT // CH2
EP2 = NC * NS * EPT
RPT = NPAD // NS
DUMMY = NPAD - 8

_SC_PARAMS = pltpu.CompilerParams(
    use_tc_tiling_on_sc=False, needs_layout_passes=False)


def _lrelu_exp(s):
    return jnp.exp(jnp.where(s >= 0, s, 0.2 * s))


def _mm1_body(x_ref, w_ref, o_ref):
    o_ref[...] = jnp.dot(x_ref[...], w_ref[...],
                         preferred_element_type=jnp.float32)


def _tc_mm1(x2, w1e):
    return pl.pallas_call(
        _mm1_body,
        grid=(40,),
        in_specs=[
            pl.BlockSpec((512, 128), lambda i: (i, 0)),
            pl.BlockSpec((128, 96), lambda i: (0, 0)),
        ],
        out_specs=pl.BlockSpec((512, 96), lambda i: (i, 0)),
        out_shape=jax.ShapeDtypeStruct((2 * NPAD, 96), jnp.float32),
    )(x2, w1e)


def _sc1_body(h0, a0, s0, d0, h1, a1, s1, d1, zr,
              out, acc0, acc1,
              sb0, sb1, sb2, db0, db1, db2,
              hg0, hg1, hg2, ag0, ag1, ag2, msg,
              isem0, isem1, isem2, gsem0, gsem1, gsem2):
    c = lax.axis_index("c")
    s = lax.axis_index("s")
    tid = c * NS + s

    sb = (sb0, sb1, sb2)
    db = (db0, db1, db2)
    hg = (hg0, hg1, hg2)
    ag = (ag0, ag1, ag2)
    isem = (isem0, isem1, isem2)
    gsem = (gsem0, gsem1, gsem2)

    pltpu.sync_copy(zr, acc0.at[pl.ds(s * RPT, RPT)])
    pltpu.sync_copy(zr, acc1.at[pl.ds(s * RPT, RPT)])
    plsc.subcore_barrier()

    iota = lax.iota(jnp.int32, 16)
    eoff2 = lax.shift_right_logical(iota, 3)
    pat0 = [2 * part + eoff2 for part in range(4)]
    pat1 = [p + 8 for p in pat0]
    swap8 = jnp.bitwise_xor(iota, 8)
    maskf = jnp.where(iota < 8, 1.0, 0.0).astype(jnp.float32)

    def perm(v, idx):
        return v.at[idx].get(mode="promise_in_bounds")

    for hT, aT, sT, dT, acc in ((h0, a0, s0, d0, acc0),
                                (h1, a1, s1, d1, acc1)):
        def issue_idx(x, q, sT=sT, dT=dT):
            base = pl.multiple_of(tid * EPT + x * CH1, 8)
            pltpu.async_copy(sT.at[pl.ds(base, CH1)], sb[q], isem[q])
            pltpu.async_copy(dT.at[pl.ds(base, CH1)], db[q], isem[q])

        def launch_gather(q, sT=sT, dT=dT, hT=hT, aT=aT):
            pltpu.make_async_copy(sT.at[pl.ds(0, CH1)], sb[q], isem[q]).wait()
            pltpu.make_async_copy(dT.at[pl.ds(0, CH1)], db[q], isem[q]).wait()
            pltpu.async_copy(hT.at[sb[q]], hg[q], gsem[q])
            pltpu.async_copy(aT.at[db[q]], ag[q], gsem[q])

        def compute(q, hT=hT, aT=aT, acc=acc):
            pltpu.make_async_copy(hT.at[sb[q]], hg[q], gsem[q]).wait()
            pltpu.make_async_copy(aT.at[db[q]], ag[q], gsem[q]).wait()
            hgp, agp = hg[q], ag[q]

            def mr(j4, _):
                for u in range(4):
                    j = j4 * 4 + u
                    r0 = 2 * j
                    r1 = 2 * j + 1
                    v0 = hgp[r0, pl.ds(64, 16)]
                    v1 = hgp[r1, pl.ds(64, 16)]
                    a0v = agp[r0, pl.ds(0, 16)]
                    a1v = agp[r1, pl.ds(0, 16)]
                    asel = jnp.where(iota < 8, v0, v1)
                    adsel = jnp.where(iota < 8, a0v, a1v)
                    w2 = _lrelu_exp(asel + adsel)
                    for part in range(4):
                        msg[r0, pl.ds(part * 16, 16)] = (
                            hgp[r0, pl.ds(part * 16, 16)] * perm(w2, pat0[part]))
                        msg[r1, pl.ds(part * 16, 16)] = (
                            hgp[r1, pl.ds(part * 16, 16)] * perm(w2, pat1[part]))
                    msg[r0, pl.ds(64, 16)] = w2 * maskf
                    msg[r1, pl.ds(64, 16)] = perm(w2, swap8) * maskf
                return ()

            lax.fori_loop(0, CH1 // 8, mr, ())
            pltpu.sync_copy(msg, acc.at[db[q]], add=True)

        issue_idx(0, 0)
        issue_idx(1, 1)
        launch_gather(0)
        launch_gather(1)
        issue_idx(2, 2)

        def step(k, _):
            for q in range(3):
                x = 3 * k + q

                @pl.when(x + 2 < NCH1)
                def _(q=q):
                    launch_gather((q + 2) % 3)

                @pl.when(x < NCH1)
                def _(q=q):
                    compute(q)

                @pl.when(x + 3 < NCH1)
                def _(x=x, q=q):
                    issue_idx(x + 3, q)

            return ()

        lax.fori_loop(0, (NCH1 + 2) // 3, step, ())

    plsc.subcore_barrier()
    pltpu.sync_copy(acc0.at[pl.ds(s * RPT, RPT)],
                    out.at[c, 0, pl.ds(s * RPT, RPT)])
    pltpu.sync_copy(acc1.at[pl.ds(s * RPT, RPT)],
                    out.at[c, 1, pl.ds(s * RPT, RPT)])


def _sc_layer1(h0, a0, s0, d0, h1, a1, s1, d1, zr):
    mesh = plsc.VectorSubcoreMesh(core_axis_name="c", subcore_axis_name="s",
                                  num_cores=NC, num_subcores=NS)
    f = pl.kernel(
        _sc1_body,
        out_type=jax.ShapeDtypeStruct((NC, 2, NPAD, 80), jnp.float32),
        mesh=mesh,
        compiler_params=_SC_PARAMS,
        scratch_types=(
            [pltpu.VMEM_SHARED((NPAD, 80), jnp.float32)] * 2
            + [pltpu.VMEM((CH1,), jnp.int32)] * 6
            + [pltpu.VMEM((CH1, 80), jnp.float32)] * 3
            + [pltpu.VMEM((CH1, 16), jnp.float32)] * 3
            + [pltpu.VMEM((CH1, 80), jnp.float32)]
            + [pltpu.SemaphoreType.DMA] * 6
        ),
    )
    return f(h0, a0, s0, d0, h1, a1, s1, d1, zr)


def _tc2_body(p0, p1, he, ad, b1r, rr, w2e, o_ref):
    ssum = p0[...] + p1[...]
    msg = ssum[:, 0:64]
    den = ssum[:, 64:72]
    asrc = he[:, 64:72]
    adst = ad[:, 0:8]
    wself = _lrelu_exp(asrc + adst)
    h1 = he[:, 0:64]
    wexp = jnp.dot(wself, rr[...], preferred_element_type=jnp.float32)
    dexp = jnp.dot(den + wself, rr[...], preferred_element_type=jnp.float32)
    out1 = (msg + h1 * wexp) / dexp + b1r[0:1, :]
    act = jnp.where(out1 > 0, out1, jnp.exp(out1) - 1.0)
    o_ref[...] = jnp.dot(act, w2e[...], preferred_element_type=jnp.float32)


def _tc_combine2(p0, p1, he, ad, b1r, rr, w2e):
    return pl.pallas_call(
        _tc2_body,
        grid=(40,),
        in_specs=[
            pl.BlockSpec((512, 80), lambda i: (i, 0)),
            pl.BlockSpec((512, 80), lambda i: (i, 0)),
            pl.BlockSpec((512, 80), lambda i: (i, 0)),
            pl.BlockSpec((512, 16), lambda i: (i, 0)),
            pl.BlockSpec((8, 64), lambda i: (0, 0)),
            pl.BlockSpec((8, 64), lambda i: (0, 0)),
            pl.BlockSpec((64, 32), lambda i: (0, 0)),
        ],
        out_specs=pl.BlockSpec((512, 32), lambda i: (i, 0)),
        out_shape=jax.ShapeDtypeStruct((2 * NPAD, 32), jnp.float32),
    )(p0, p1, he, ad, b1r, rr, w2e)


def _sc2_body(h0, a0, s0, d0, h1, a1, s1, d1, zr,
              out, acc0, acc1, av0, av1,
              sb0, sb1, sb2, sb3, db0, db1, db2, db3,
              hg0, hg1, hg2, hg3, msg, wbuf,
              isem0, isem1, isem2, isem3, gsem0, gsem1, gsem2, gsem3):
    c = lax.axis_index("c")
    s = lax.axis_index("s")
    tid = c * NS + s

    sb = (sb0, sb1, sb2, sb3)
    db = (db0, db1, db2, db3)
    hg = (hg0, hg1, hg2, hg3)
    isem = (isem0, isem1, isem2, isem3)
    gsem = (gsem0, gsem1, gsem2, gsem3)

    pltpu.sync_copy(zr, acc0.at[pl.ds(s * RPT, RPT)])
    pltpu.sync_copy(zr, acc1.at[pl.ds(s * RPT, RPT)])
    pltpu.sync_copy(a0, av0)
    pltpu.sync_copy(a1, av1)
    plsc.subcore_barrier()

    iota = lax.iota(jnp.int32, 16)
    c16 = jnp.full((16,), 16, jnp.int32)
    mask0 = jnp.where(iota < 1, 1.0, 0.0).astype(jnp.float32)
    cfull = [jnp.full((16,), u, jnp.int32) for u in range(16)]

    def perm(v, idx):
        return v.at[idx].get(mode="promise_in_bounds")

    for hT, sT, dT, av, acc in ((h0, s0, d0, av0, acc0),
                                (h1, s1, d1, av1, acc1)):
        def issue_idx(x, q, sT=sT, dT=dT):
            base = pl.multiple_of(tid * EPT + x * CH2, 8)
            pltpu.async_copy(sT.at[pl.ds(base, CH2)], sb[q], isem[q])
            pltpu.async_copy(dT.at[pl.ds(base, CH2)], db[q], isem[q])

        def launch_gather(q, sT=sT, dT=dT, hT=hT):
            pltpu.make_async_copy(sT.at[pl.ds(0, CH2)], sb[q], isem[q]).wait()
            pltpu.make_async_copy(dT.at[pl.ds(0, CH2)], db[q], isem[q]).wait()
            pltpu.async_copy(hT.at[sb[q]], hg[q], gsem[q])

        def compute(q, hT=hT, av=av, acc=acc):
            pltpu.make_async_copy(hT.at[sb[q]], hg[q], gsem[q]).wait()
            hgp, dbp = hg[q], db[q]

            def wj(j, _):
                erow = iota + 16 * j
                asrc = plsc.load_gather(hgp, [erow, c16])
                didx = dbp[pl.ds(j * 16, 16)]
                adst = plsc.load_gather(av, [didx])
                wbuf[pl.ds(j * 16, 16)] = _lrelu_exp(asrc + adst)
                return ()

            lax.fori_loop(0, CH2 // 16, wj, ())

            def mr(r16, _):
                wvv = wbuf[pl.ds(r16 * 16, 16)]
                for u in range(16):
                    r = r16 * 16 + u
                    wv = perm(wvv, cfull[u])
                    msg[r, pl.ds(0, 16)] = hgp[r, pl.ds(0, 16)] * wv
                    msg[r, pl.ds(16, 16)] = wv * mask0
                return ()

            lax.fori_loop(0, CH2 // 16, mr, ())
            pltpu.sync_copy(msg, acc.at[dbp], add=True)

        issue_idx(0, 0)
        issue_idx(1, 1)
        launch_gather(0)
        launch_gather(1)
        issue_idx(2, 2)
        issue_idx(3, 3)

        def step(k, _):
            for q in range(4):
                x = 4 * k + q

                @pl.when(x + 2 < NCH2)
                def _(q=q):
                    launch_gather((q + 2) % 4)

                compute(q)

                @pl.when(x + 4 < NCH2)
                def _(x=x, q=q):
                    issue_idx(x + 4, q)

            return ()

        lax.fori_loop(0, NCH2 // 4, step, ())

    plsc.subcore_barrier()
    pltpu.sync_copy(acc0.at[pl.ds(s * RPT, RPT)],
                    out.at[c, 0, pl.ds(s * RPT, RPT)])
    pltpu.sync_copy(acc1.at[pl.ds(s * RPT, RPT)],
                    out.at[c, 1, pl.ds(s * RPT, RPT)])


def _sc_layer2(h0, a0, s0, d0, h1, a1, s1, d1, zr):
    mesh = plsc.VectorSubcoreMesh(core_axis_name="c", subcore_axis_name="s",
                                  num_cores=NC, num_subcores=NS)
    f = pl.kernel(
        _sc2_body,
        out_type=jax.ShapeDtypeStruct((NC, 2, NPAD, 32), jnp.float32),
        mesh=mesh,
        compiler_params=_SC_PARAMS,
        scratch_types=(
            [pltpu.VMEM_SHARED((NPAD, 32), jnp.float32)] * 2
            + [pltpu.VMEM((NPAD,), jnp.float32)] * 2
            + [pltpu.VMEM((CH2,), jnp.int32)] * 8
            + [pltpu.VMEM((CH2, 32), jnp.float32)] * 4
            + [pltpu.VMEM((CH2, 32), jnp.float32)]
            + [pltpu.VMEM((CH2,), jnp.float32)]
            + [pltpu.SemaphoreType.DMA] * 8
        ),
    )
    return f(h0, a0, s0, d0, h1, a1, s1, d1, zr)


def _tc3_body(qy0, qy1, qz0, qz1, hy, hz, b2r, ly_ref, lz_ref, omc_ref):
    def node_out(q0, q1, he):
        ssum = q0[...] + q1[...]
        msg = ssum[:, 0:16]
        den = ssum[:, 16:17]
        asrc = he[:, 16:17]
        adst = he[:, 17:18]
        wself = _lrelu_exp(asrc + adst)
        h2 = he[:, 0:16]
        return (msg + h2 * wself) / (den + wself) + b2r[0:1, :]

    y = node_out(qy0, qy1, hy)
    z = node_out(qz0, qz1, hz)

    def logsm(v):
        m = jnp.max(v, axis=1, keepdims=True)
        return v - m - jnp.log(jnp.sum(jnp.exp(v - m), axis=1, keepdims=True))

    ly_ref[...] = logsm(y)
    lz_ref[...] = logsm(z)
    dot = jnp.sum(y * z, axis=1, keepdims=True)
    ny = jnp.maximum(jnp.sqrt(jnp.sum(y * y, axis=1, keepdims=True)), 1e-8)
    nz = jnp.maximum(jnp.sqrt(jnp.sum(z * z, axis=1, keepdims=True)), 1e-8)
    omc_ref[...] = 1.0 - dot / (ny * nz)


def _tc_final(qy0, qy1, qz0, qz1, hy, hz, b2r):
    return pl.pallas_call(
        _tc3_body,
        grid=(25,),
        in_specs=[
            pl.BlockSpec((400, 32), lambda i: (i, 0)),
            pl.BlockSpec((400, 32), lambda i: (i, 0)),
            pl.BlockSpec((400, 32), lambda i: (i, 0)),
            pl.BlockSpec((400, 32), lambda i: (i, 0)),
            pl.BlockSpec((400, 32), lambda i: (i, 0)),
            pl.BlockSpec((400, 32), lambda i: (i, 0)),
            pl.BlockSpec((8, 16), lambda i: (0, 0)),
        ],
        out_specs=[
            pl.BlockSpec((400, 16), lambda i: (i, 0)),
            pl.BlockSpec((400, 16), lambda i: (i, 0)),
            pl.BlockSpec((400, 1), lambda i: (i, 0)),
        ],
        out_shape=[
            jax.ShapeDtypeStruct((N, 16), jnp.float32),
            jax.ShapeDtypeStruct((N, 16), jnp.float32),
            jax.ShapeDtypeStruct((N, 1), jnp.float32),
        ],
    )(qy0, qy1, qz0, qz1, hy, hz, b2r)


@jax.jit
def kernel(x, edge_index, trans_x, trans_edge_index,
           W1, a1_src, a1_dst, b1, W2, a2_src, a2_dst, b2):
    f32 = jnp.float32
    ar64 = jnp.arange(64)
    A1s = jnp.zeros((64, 8), f32).at[ar64, ar64 // 8].set(a1_src.reshape(64))
    A1d = jnp.zeros((64, 8), f32).at[ar64, ar64 // 8].set(a1_dst.reshape(64))
    w1s = W1 @ A1s
    w1d = W1 @ A1d
    w1e = jnp.concatenate([W1, w1s, w1s, w1d, w1d], axis=1)

    zrow = jnp.zeros((NPAD - N, 128), f32)
    x2 = jnp.concatenate([x, zrow, trans_x, zrow], axis=0)
    y1 = _tc_mm1(x2, w1e)
    hext = y1[:, 0:80]
    adstt = y1[:, 80:96]

    epad = jnp.full((2, EP2 - E), DUMMY, jnp.int32)
    ei0 = jnp.concatenate([edge_index, epad], axis=1)
    ei1 = jnp.concatenate([trans_edge_index, epad], axis=1)
    s0, d0 = ei0[0], ei0[1]
    s1, d1 = ei1[0], ei1[1]

    zr1 = jnp.zeros((RPT, 80), f32)
    p = _sc_layer1(hext[:NPAD], adstt[:NPAD], s0, d0,
                   hext[NPAD:], adstt[NPAD:], s1, d1, zr1)

    w2e = jnp.concatenate(
        [W2, W2 @ a2_src.T, W2 @ a2_dst.T, jnp.zeros((64, 14), f32)], axis=1)
    rr = jnp.zeros((8, 64), f32).at[ar64 // 8, ar64].set(1.0)
    b1r = jnp.broadcast_to(b1, (8, 64))
    h2e = _tc_combine2(p[0].reshape(2 * NPAD, 80), p[1].reshape(2 * NPAD, 80),
                       hext, adstt, b1r, rr, w2e)

    av = h2e[:, 17]
    zr2 = jnp.zeros((RPT, 32), f32)
    q = _sc_layer2(h2e[:NPAD], av[:NPAD], s0, d0,
                   h2e[NPAD:], av[NPAD:], s1, d1, zr2)

    b2r = jnp.broadcast_to(b2, (8, 16))
    ly, lz, omc = _tc_final(q[0, 0], q[1, 0], q[0, 1], q[1, 1],
                            h2e[:NPAD], h2e[NPAD:], b2r)
    return (ly, omc.reshape(N), lz, ly, ly)
